# baseline (speedup 1.0000x reference)
import functools

import jax
import jax.numpy as jnp
from jax import lax
from jax.experimental import pallas as pl
from jax.experimental.pallas import tpu as pltpu
from jax.experimental.pallas import tpu_sc as plsc

R = 16384
C = 1000
NC, NS, L = 2, 16, 16
NW = NC * NS
ROWS_W = R // NW
NT = ROWS_W // L
TW = L * C

_mesh = plsc.VectorSubcoreMesh(core_axis_name="c", subcore_axis_name="s")


@functools.partial(
    pl.kernel,
    out_type=jax.ShapeDtypeStruct((R * C,), jnp.float32),
    mesh=_mesh,
    scratch_types=[
        pltpu.VMEM((TW,), jnp.float32),
        pltpu.VMEM((TW,), jnp.float32),
    ],
    compiler_params=pltpu.CompilerParams(needs_layout_passes=False),
)
def _onehot_sc(x_hbm, out_hbm, xv, ov):
    wid = lax.axis_index("s") * NC + lax.axis_index("c")
    lane_base = lax.iota(jnp.int32, L) * C

    def zero_body(i, _):
        ov[pl.ds(i * L, L)] = jnp.zeros((L,), jnp.float32)
        return 0

    lax.fori_loop(0, TW // L, zero_body, 0)

    def tile_body(t, prev_hot):
        base = (wid * NT + t) * TW
        pltpu.sync_copy(x_hbm.at[pl.ds(base, TW)], xv)

        def amax_body(j, carry):
            mv, mo, off = carry
            col = plsc.load_gather(xv, [off])
            pred = col > mv
            mv = jnp.where(pred, col, mv)
            mo = jnp.where(pred, off, mo)
            return mv, mo, off + 1

        minf = jnp.full((L,), -jnp.inf, jnp.float32)
        _, mo, _ = lax.fori_loop(0, C, amax_body, (minf, lane_base, lane_base))

        plsc.store_scatter(ov, [prev_hot], jnp.zeros((L,), jnp.float32))
        plsc.store_scatter(ov, [mo], jnp.ones((L,), jnp.float32))
        pltpu.sync_copy(ov, out_hbm.at[pl.ds(base, TW)])
        return mo

    lax.fori_loop(0, NT, tile_body, lane_base)


def kernel(x):
    out = _onehot_sc(x.reshape(-1))
    return out.reshape(R, C)


# speedup vs baseline: 1.4704x; 1.4704x over previous
import functools

import jax
import jax.numpy as jnp
from jax import lax
from jax.experimental import pallas as pl
from jax.experimental.pallas import tpu as pltpu
from jax.experimental.pallas import tpu_sc as plsc

R = 16384
C = 1000
NC, NS, L = 2, 16, 16
NW = NC * NS
ROWS_W = R // NW
NT = ROWS_W // L
NPAIR = NT // 2
TW = L * C
U = 10

_mesh = plsc.VectorSubcoreMesh(core_axis_name="c", subcore_axis_name="s")


@functools.partial(
    pl.kernel,
    out_type=jax.ShapeDtypeStruct((R * C,), jnp.float32),
    mesh=_mesh,
    scratch_types=[
        pltpu.VMEM((TW,), jnp.float32),
        pltpu.VMEM((TW,), jnp.float32),
        pltpu.VMEM((TW,), jnp.float32),
        pltpu.VMEM((TW,), jnp.float32),
        pltpu.SemaphoreType.DMA,
        pltpu.SemaphoreType.DMA,
        pltpu.SemaphoreType.DMA,
        pltpu.SemaphoreType.DMA,
    ],
    compiler_params=pltpu.CompilerParams(needs_layout_passes=False),
)
def _onehot_sc(x_hbm, out_hbm, xa, xb, oa, ob, sxa, sxb, soa, sob):
    wid = lax.axis_index("s") * NC + lax.axis_index("c")
    lane_base = lax.iota(jnp.int32, L) * C
    zeros = jnp.zeros((L,), jnp.float32)
    ones = jnp.ones((L,), jnp.float32)
    minf = jnp.full((L,), -jnp.inf, jnp.float32)
    wbase = wid * NT * TW

    def zero_body(i, _):
        oa[pl.ds(i * L, L)] = zeros
        ob[pl.ds(i * L, L)] = zeros
        return 0

    lax.fori_loop(0, TW // L, zero_body, 0)

    pltpu.async_copy(x_hbm.at[pl.ds(wbase, TW)], xa, sxa)
    pltpu.async_copy(x_hbm.at[pl.ds(wbase + TW, TW)], xb, sxb)

    def half(i, t, xv, ov, sx, so, prev_hot):
        base = wbase + t * TW
        pltpu.make_async_copy(x_hbm.at[pl.ds(base, TW)], xv, sx).wait()

        def amax_body(jj, carry):
            mv, mo, off = carry
            for _ in range(U):
                col = plsc.load_gather(xv, [off])
                pred = col > mv
                mv = jnp.where(pred, col, mv)
                mo = jnp.where(pred, off, mo)
                off = off + 1
            return mv, mo, off

        _, mo, _ = lax.fori_loop(0, C // U, amax_body,
                                 (minf, lane_base, lane_base))

        @pl.when(i > 0)
        def _wait_out():
            pltpu.make_async_copy(ov, out_hbm.at[pl.ds(base, TW)], so).wait()

        plsc.store_scatter(ov, [prev_hot], zeros)
        plsc.store_scatter(ov, [mo], ones)
        pltpu.async_copy(ov, out_hbm.at[pl.ds(base, TW)], so)

        @pl.when(i < NPAIR - 1)
        def _next_x():
            pltpu.async_copy(x_hbm.at[pl.ds(base + 2 * TW, TW)], xv, sx)

        return mo

    def pair_body(i, carry):
        pa, pb = carry
        pa = half(i, 2 * i, xa, oa, sxa, soa, pa)
        pb = half(i, 2 * i + 1, xb, ob, sxb, sob, pb)
        return (pa, pb)

    lax.fori_loop(0, NPAIR, pair_body, (lane_base, lane_base))

    pltpu.make_async_copy(oa, out_hbm.at[pl.ds(0, TW)], soa).wait()
    pltpu.make_async_copy(ob, out_hbm.at[pl.ds(0, TW)], sob).wait()


def kernel(x):
    out = _onehot_sc(x.reshape(-1))
    return out.reshape(R, C)


# speedup vs baseline: 1.5023x; 1.0217x over previous
import functools

import jax
import jax.numpy as jnp
from jax import lax
from jax.experimental import pallas as pl
from jax.experimental.pallas import tpu as pltpu
from jax.experimental.pallas import tpu_sc as plsc

R = 16384
C = 1000
NC, NS, L = 2, 16, 16
NW = NC * NS
ROWS_W = R // NW
NT = ROWS_W // L
NPAIR = NT // 2
TW = L * C
A = 4
SEG = C // A
Q = 5

_mesh = plsc.VectorSubcoreMesh(core_axis_name="c", subcore_axis_name="s")


@functools.partial(
    pl.kernel,
    out_type=jax.ShapeDtypeStruct((R * C,), jnp.float32),
    mesh=_mesh,
    scratch_types=[
        pltpu.VMEM((TW,), jnp.float32),
        pltpu.VMEM((TW,), jnp.float32),
        pltpu.VMEM((TW,), jnp.float32),
        pltpu.VMEM((TW,), jnp.float32),
        pltpu.SemaphoreType.DMA,
        pltpu.SemaphoreType.DMA,
        pltpu.SemaphoreType.DMA,
        pltpu.SemaphoreType.DMA,
    ],
    compiler_params=pltpu.CompilerParams(needs_layout_passes=False),
)
def _onehot_sc(x_hbm, out_hbm, xa, xb, oa, ob, sxa, sxb, soa, sob):
    wid = lax.axis_index("s") * NC + lax.axis_index("c")
    lane_base = lax.iota(jnp.int32, L) * C
    zeros = jnp.zeros((L,), jnp.float32)
    ones = jnp.ones((L,), jnp.float32)
    minf = jnp.full((L,), -jnp.inf, jnp.float32)
    wbase = wid * NT * TW

    def zero_body(i, _):
        oa[pl.ds(i * L, L)] = zeros
        ob[pl.ds(i * L, L)] = zeros
        return 0

    lax.fori_loop(0, TW // L, zero_body, 0)

    pltpu.async_copy(x_hbm.at[pl.ds(wbase, TW)], xa, sxa)
    pltpu.async_copy(x_hbm.at[pl.ds(wbase + TW, TW)], xb, sxb)

    def half(i, t, xv, ov, sx, so, prev_hot):
        base = wbase + t * TW
        pltpu.make_async_copy(x_hbm.at[pl.ds(base, TW)], xv, sx).wait()

        def amax_body(jj, carry):
            mvs, mos, off = carry
            mvs, mos = list(mvs), list(mos)
            for q in range(Q):
                for a in range(A):
                    o = off + (a * SEG + q)
                    col = plsc.load_gather(xv, [o])
                    pred = col > mvs[a]
                    mos[a] = jnp.where(pred, o, mos[a])
                    mvs[a] = jnp.maximum(mvs[a], col)
            return tuple(mvs), tuple(mos), off + Q

        init_mos = tuple(lane_base + a * SEG for a in range(A))
        mvs, mos, _ = lax.fori_loop(0, SEG // Q, amax_body,
                                    ((minf,) * A, init_mos, lane_base))
        mv, mo = mvs[0], mos[0]
        for a in range(1, A):
            pred = mvs[a] > mv
            mo = jnp.where(pred, mos[a], mo)
            mv = jnp.where(pred, mvs[a], mv)

        @pl.when(i > 0)
        def _wait_out():
            pltpu.make_async_copy(ov, out_hbm.at[pl.ds(base, TW)], so).wait()

        plsc.store_scatter(ov, [prev_hot], zeros)
        plsc.store_scatter(ov, [mo], ones)
        pltpu.async_copy(ov, out_hbm.at[pl.ds(base, TW)], so)

        @pl.when(i < NPAIR - 1)
        def _next_x():
            pltpu.async_copy(x_hbm.at[pl.ds(base + 2 * TW, TW)], xv, sx)

        return mo

    def pair_body(i, carry):
        pa, pb = carry
        pa = half(i, 2 * i, xa, oa, sxa, soa, pa)
        pb = half(i, 2 * i + 1, xb, ob, sxb, sob, pb)
        return (pa, pb)

    lax.fori_loop(0, NPAIR, pair_body, (lane_base, lane_base))

    pltpu.make_async_copy(oa, out_hbm.at[pl.ds(0, TW)], soa).wait()
    pltpu.make_async_copy(ob, out_hbm.at[pl.ds(0, TW)], sob).wait()


def kernel(x):
    out = _onehot_sc(x.reshape(-1))
    return out.reshape(R, C)
